# baseline (speedup 1.0000x reference)
import functools

import jax
import jax.numpy as jnp
from jax import lax
from jax.experimental import pallas as pl
from jax.experimental.pallas import tpu as pltpu
from jax.experimental.pallas import tpu_sc as plsc

_B, _T, _N, _D = 4, 8, 1024, 128
_C, _L = 2, 2
_E = 32768
_P = 4


def _build_adjacency(edge_bwd, edge_fwd):
    info = plsc.get_sparse_core_info()
    n_sub = info.num_subcores
    lanes = info.num_lanes
    e_per_tile = _E // n_sub
    rows_per_tile = (_N * _N) // n_sub

    zeros_hbm = jnp.zeros((rows_per_tile,), jnp.float32)
    ones_hbm = jnp.ones((e_per_tile,), jnp.float32)

    mesh = plsc.VectorSubcoreMesh(core_axis_name="c", subcore_axis_name="s")

    @functools.partial(
        pl.kernel,
        mesh=mesh,
        out_type=[
            jax.ShapeDtypeStruct((2, _N * _N), jnp.float32),
            jax.ShapeDtypeStruct((2, _N), jnp.float32),
        ],
        scratch_types=[
            pltpu.VMEM((e_per_tile,), jnp.int32),
            pltpu.VMEM((e_per_tile,), jnp.int32),
            pltpu.VMEM((e_per_tile,), jnp.int32),
            pltpu.VMEM((e_per_tile,), jnp.float32),
            pltpu.VMEM_SHARED((_N * _N + _N,), jnp.float32),
        ],
    )
    def build(e_bwd, e_fwd, zeros_in, ones_in, a_out, deg_out, src_v, dst_v,
              idx_v, ones_v, a_sh):
        cid = lax.axis_index("c")
        sid = lax.axis_index("s")
        row0 = sid * rows_per_tile
        ebase = sid * e_per_tile

        pltpu.sync_copy(zeros_in, a_sh.at[pl.ds(row0, rows_per_tile)])
        pltpu.sync_copy(ones_in, ones_v)

        @pl.when(sid == 0)
        def _():
            pltpu.sync_copy(zeros_in.at[pl.ds(0, _N)],
                            a_sh.at[pl.ds(_N * _N, _N)])

        def scatter_edges(e_ref):
            pltpu.sync_copy(e_ref.at[0, pl.ds(ebase, e_per_tile)], src_v)
            pltpu.sync_copy(e_ref.at[1, pl.ds(ebase, e_per_tile)], dst_v)

            def body(i, _):
                sl = pl.ds(i * lanes, lanes)
                idx_v[sl] = dst_v[sl] * _N + src_v[sl]
                src_v[sl] = dst_v[sl] + (_N * _N)
                return ()

            lax.fori_loop(0, e_per_tile // lanes, body, ())
            plsc.subcore_barrier()
            pltpu.sync_copy(ones_v, a_sh.at[idx_v], add=True)
            pltpu.sync_copy(ones_v, a_sh.at[src_v], add=True)
            plsc.subcore_barrier()

        @pl.when(cid == 0)
        def _():
            scatter_edges(e_bwd)
            pltpu.sync_copy(a_sh.at[pl.ds(row0, rows_per_tile)],
                            a_out.at[0, pl.ds(row0, rows_per_tile)])

            @pl.when(sid == 0)
            def _():
                pltpu.sync_copy(a_sh.at[pl.ds(_N * _N, _N)], deg_out.at[0])

        @pl.when(cid == 1)
        def _():
            scatter_edges(e_fwd)
            pltpu.sync_copy(a_sh.at[pl.ds(row0, rows_per_tile)],
                            a_out.at[1, pl.ds(row0, rows_per_tile)])

            @pl.when(sid == 0)
            def _():
                pltpu.sync_copy(a_sh.at[pl.ds(_N * _N, _N)], deg_out.at[1])

    a_flat, deg = build(edge_bwd, edge_fwd, zeros_hbm, ones_hbm)
    return a_flat.reshape(2, _N, _N), deg


def _bf(x):
    return x.astype(jnp.bfloat16)


def _tc_body(x_ref, a_ref, deg_ref, wg_ref, bg_ref, wc_ref, wl_ref, ob_ref,
             out_ref):
    acc = None
    for half in range(2):
        a = a_ref[half]
        inv = 1.0 / jnp.maximum(deg_ref[half], 1.0)
        h_in = x_ref[0, half]
        m0 = jnp.dot(a, _bf(h_in), preferred_element_type=jnp.float32)
        g0 = m0 * inv + h_in
        z = None
        for c in range(_C):
            h1 = jnp.maximum(
                jnp.dot(_bf(g0), wg_ref[c, half, 0],
                        preferred_element_type=jnp.float32) + bg_ref[c, half, 0],
                0.0)
            m1 = jnp.dot(a, _bf(h1), preferred_element_type=jnp.float32)
            g1 = m1 * inv + h1
            h2 = jnp.maximum(
                jnp.dot(_bf(g1), wg_ref[c, half, 1],
                        preferred_element_type=jnp.float32) + bg_ref[c, half, 1],
                0.0)
            zc = jnp.dot(_bf(h2), wl_ref[c], preferred_element_type=jnp.float32)
            z = zc if z is None else z + zc
        y = jnp.dot(wc_ref[:, half * _N:(half + 1) * _N], _bf(z),
                    preferred_element_type=jnp.float32)
        acc = y if acc is None else acc + y
    out_ref[0, 0] = acc + ob_ref[...]


def kernel(inputs, edge_index_bwd, edge_index_fwd, W_gcn, b_gcn, W_conv,
           b_conv, W_lin, b_lin):
    a2, deg2 = _build_adjacency(edge_index_bwd, edge_index_fwd)
    a2 = a2.astype(jnp.bfloat16)
    deg2 = deg2.reshape(2, _N, 1)

    wg = W_gcn.astype(jnp.bfloat16)
    wc = W_conv.astype(jnp.bfloat16)
    wl = W_lin.reshape(_C, _D, _D).astype(jnp.bfloat16)
    bg = b_gcn.reshape(_C, 2, _L, 1, _D)
    out_bias = b_conv[:, None] * jnp.sum(W_lin, axis=0)[None, :] + b_lin[None, :]

    grid = (_B, _P)
    out = pl.pallas_call(
        _tc_body,
        grid=grid,
        in_specs=[
            pl.BlockSpec((1, 2, _N, _D), lambda b, p: (b, p, 0, 0)),
            pl.BlockSpec((2, _N, _N), lambda b, p: (0, 0, 0)),
            pl.BlockSpec((2, _N, 1), lambda b, p: (0, 0, 0)),
            pl.BlockSpec((_C, 2, _L, _D, _D), lambda b, p: (0, 0, 0, 0, 0)),
            pl.BlockSpec((_C, 2, _L, 1, _D), lambda b, p: (0, 0, 0, 0, 0)),
            pl.BlockSpec((_N, 2 * _N), lambda b, p: (0, 0)),
            pl.BlockSpec((_C, _D, _D), lambda b, p: (0, 0, 0)),
            pl.BlockSpec((_N, _D), lambda b, p: (0, 0)),
        ],
        out_specs=pl.BlockSpec((1, 1, _N, _D), lambda b, p: (b, p, 0, 0)),
        out_shape=jax.ShapeDtypeStruct((_B, _P, _N, _D), jnp.float32),
        compiler_params=pltpu.CompilerParams(
            dimension_semantics=("parallel", "parallel"),
            vmem_limit_bytes=100 * 1024 * 1024,
        ),
    )(inputs, a2, deg2, W_gcn.astype(jnp.bfloat16), bg, wc, wl, out_bias)
    return out


# speedup vs baseline: 1.6646x; 1.6646x over previous
import functools

import jax
import jax.numpy as jnp
from jax import lax
from jax.experimental import pallas as pl
from jax.experimental.pallas import tpu as pltpu
from jax.experimental.pallas import tpu_sc as plsc

_B, _T, _N, _D = 4, 8, 1024, 128
_C, _L = 2, 2
_E = 32768
_P = 4


def _build_adjacency(edge_bwd, edge_fwd):
    info = plsc.get_sparse_core_info()
    n_sub = info.num_subcores
    lanes = info.num_lanes
    e_per_tile = _E // n_sub
    rows_per_tile = (_N * _N) // n_sub

    zeros_hbm = jnp.zeros((rows_per_tile,), jnp.float32)
    ones_hbm = jnp.ones((e_per_tile,), jnp.float32)

    mesh = plsc.VectorSubcoreMesh(core_axis_name="c", subcore_axis_name="s")

    @functools.partial(
        pl.kernel,
        mesh=mesh,
        out_type=[
            jax.ShapeDtypeStruct((2, _N * _N), jnp.float32),
            jax.ShapeDtypeStruct((2, _N), jnp.float32),
        ],
        scratch_types=[
            pltpu.VMEM((e_per_tile,), jnp.int32),
            pltpu.VMEM((e_per_tile,), jnp.int32),
            pltpu.VMEM((e_per_tile,), jnp.int32),
            pltpu.VMEM((e_per_tile,), jnp.float32),
            pltpu.VMEM_SHARED((_N * _N + _N,), jnp.float32),
        ],
    )
    def build(e_bwd, e_fwd, zeros_in, ones_in, a_out, deg_out, src_v, dst_v,
              idx_v, ones_v, a_sh):
        cid = lax.axis_index("c")
        sid = lax.axis_index("s")
        row0 = sid * rows_per_tile
        ebase = sid * e_per_tile

        pltpu.sync_copy(zeros_in, a_sh.at[pl.ds(row0, rows_per_tile)])
        pltpu.sync_copy(ones_in, ones_v)

        @pl.when(sid == 0)
        def _():
            pltpu.sync_copy(zeros_in.at[pl.ds(0, _N)],
                            a_sh.at[pl.ds(_N * _N, _N)])

        def scatter_edges(e_ref):
            pltpu.sync_copy(e_ref.at[0, pl.ds(ebase, e_per_tile)], src_v)
            pltpu.sync_copy(e_ref.at[1, pl.ds(ebase, e_per_tile)], dst_v)

            def body(i, _):
                sl = pl.ds(i * lanes, lanes)
                idx_v[sl] = dst_v[sl] * _N + src_v[sl]
                src_v[sl] = dst_v[sl] + (_N * _N)
                return ()

            lax.fori_loop(0, e_per_tile // lanes, body, ())
            plsc.subcore_barrier()
            pltpu.sync_copy(ones_v, a_sh.at[idx_v], add=True)
            pltpu.sync_copy(ones_v, a_sh.at[src_v], add=True)
            plsc.subcore_barrier()

        @pl.when(cid == 0)
        def _():
            scatter_edges(e_bwd)
            pltpu.sync_copy(a_sh.at[pl.ds(row0, rows_per_tile)],
                            a_out.at[0, pl.ds(row0, rows_per_tile)])

            @pl.when(sid == 0)
            def _():
                pltpu.sync_copy(a_sh.at[pl.ds(_N * _N, _N)], deg_out.at[0])

        @pl.when(cid == 1)
        def _():
            scatter_edges(e_fwd)
            pltpu.sync_copy(a_sh.at[pl.ds(row0, rows_per_tile)],
                            a_out.at[1, pl.ds(row0, rows_per_tile)])

            @pl.when(sid == 0)
            def _():
                pltpu.sync_copy(a_sh.at[pl.ds(_N * _N, _N)], deg_out.at[1])

    a_flat, deg = build(edge_bwd, edge_fwd, zeros_hbm, ones_hbm)
    return a_flat.reshape(2, _N, _N), deg


_BD = _B * _D


def _bf(x):
    return x.astype(jnp.bfloat16)


def _tc_body(x_ref, a_ref, deg_ref, wg0_ref, bg0_ref, wg1_ref, bg1_ref,
             wc_ref, wl_ref, ob_ref, out_ref):
    h = pl.program_id(1)
    a = a_ref[0]
    inv = 1.0 / jnp.maximum(deg_ref[0], 1.0)
    xh = x_ref[0]
    m0 = jnp.dot(a, _bf(xh), preferred_element_type=jnp.float32)
    g0 = m0 * inv + xh

    h1_parts = []
    for b in range(_B):
        gb = g0[:, b * _D:(b + 1) * _D]
        h1_parts.append(jnp.maximum(
            jnp.dot(_bf(gb), wg0_ref[0], preferred_element_type=jnp.float32)
            + bg0_ref[0], 0.0))
    h1cat = jnp.concatenate(
        [p[:, :_D] for p in h1_parts] + [p[:, _D:] for p in h1_parts], axis=1)

    m1 = jnp.dot(a, _bf(h1cat), preferred_element_type=jnp.float32)
    g1 = m1 * inv + h1cat

    zs = []
    for b in range(_B):
        zb = None
        for c in range(_C):
            gcb = g1[:, (c * _B + b) * _D:(c * _B + b + 1) * _D]
            h2 = jnp.maximum(
                jnp.dot(_bf(gcb), wg1_ref[c, 0],
                        preferred_element_type=jnp.float32) + bg1_ref[c, 0],
                0.0)
            zc = jnp.dot(_bf(h2), wl_ref[c], preferred_element_type=jnp.float32)
            zb = zc if zb is None else zb + zc
        zs.append(zb)
    z = jnp.concatenate(zs, axis=1)

    y = jnp.dot(wc_ref[...], _bf(z), preferred_element_type=jnp.float32)

    @pl.when(h == 0)
    def _():
        out_ref[0] = y + ob_ref[...]

    @pl.when(h == 1)
    def _():
        out_ref[0] = out_ref[0] + y


def kernel(inputs, edge_index_bwd, edge_index_fwd, W_gcn, b_gcn, W_conv,
           b_conv, W_lin, b_lin):
    a2, deg2 = _build_adjacency(edge_index_bwd, edge_index_fwd)
    a2 = a2.astype(jnp.bfloat16)
    deg2 = deg2.reshape(2, _N, 1)

    xw = inputs.transpose(1, 2, 0, 3).reshape(_T, _N, _BD)
    wg = W_gcn.astype(jnp.bfloat16)
    wg0 = jnp.concatenate([wg[0, :, 0], wg[1, :, 0]], axis=-1)
    bg0 = jnp.concatenate([b_gcn[0, :, 0], b_gcn[1, :, 0]], axis=-1)[:, None, :]
    wg1 = wg[:, :, 1]
    bg1 = b_gcn[:, :, 1][:, :, None, :]
    wc = W_conv.astype(jnp.bfloat16)
    wl = W_lin.reshape(_C, _D, _D).astype(jnp.bfloat16)
    ob = b_conv[:, None] * jnp.sum(W_lin, axis=0)[None, :] + b_lin[None, :]
    ob = jnp.tile(ob, (1, _B))

    yw = pl.pallas_call(
        _tc_body,
        grid=(_P, 2),
        in_specs=[
            pl.BlockSpec((1, _N, _BD), lambda p, h: (2 * p + h, 0, 0)),
            pl.BlockSpec((1, _N, _N), lambda p, h: (h, 0, 0)),
            pl.BlockSpec((1, _N, 1), lambda p, h: (h, 0, 0)),
            pl.BlockSpec((1, _D, 2 * _D), lambda p, h: (h, 0, 0)),
            pl.BlockSpec((1, 1, 2 * _D), lambda p, h: (h, 0, 0)),
            pl.BlockSpec((_C, 1, _D, _D), lambda p, h: (0, h, 0, 0)),
            pl.BlockSpec((_C, 1, 1, _D), lambda p, h: (0, h, 0, 0)),
            pl.BlockSpec((_N, _N), lambda p, h: (0, h)),
            pl.BlockSpec((_C, _D, _D), lambda p, h: (0, 0, 0)),
            pl.BlockSpec((_N, _BD), lambda p, h: (0, 0)),
        ],
        out_specs=pl.BlockSpec((1, _N, _BD), lambda p, h: (p, 0, 0)),
        out_shape=jax.ShapeDtypeStruct((_P, _N, _BD), jnp.float32),
        compiler_params=pltpu.CompilerParams(
            dimension_semantics=("parallel", "arbitrary"),
            vmem_limit_bytes=120 * 1024 * 1024,
        ),
    )(xw, a2, deg2, wg0, bg0, wg1, bg1, wc, wl, ob)
    return yw.reshape(_P, _N, _B, _D).transpose(2, 0, 1, 3)


# speedup vs baseline: 1.8593x; 1.1169x over previous
import functools

import jax
import jax.numpy as jnp
from jax import lax
from jax.experimental import pallas as pl
from jax.experimental.pallas import tpu as pltpu
from jax.experimental.pallas import tpu_sc as plsc

_B, _T, _N, _D = 4, 8, 1024, 128
_C, _L = 2, 2
_E = 32768
_P = 4


def _build_adjacency(edge_bwd, edge_fwd):
    info = plsc.get_sparse_core_info()
    n_sub = info.num_subcores
    lanes = info.num_lanes
    e_per_tile = _E // n_sub
    rows_per_tile = (_N * _N) // n_sub

    zeros_hbm = jnp.zeros((rows_per_tile,), jnp.float32)
    ones_hbm = jnp.ones((e_per_tile,), jnp.float32)

    mesh = plsc.VectorSubcoreMesh(core_axis_name="c", subcore_axis_name="s")

    @functools.partial(
        pl.kernel,
        mesh=mesh,
        out_type=[
            jax.ShapeDtypeStruct((2, _N * _N), jnp.float32),
            jax.ShapeDtypeStruct((2, _N), jnp.float32),
        ],
        scratch_types=[
            pltpu.VMEM((e_per_tile,), jnp.int32),
            pltpu.VMEM((e_per_tile,), jnp.int32),
            pltpu.VMEM((e_per_tile,), jnp.int32),
            pltpu.VMEM((e_per_tile,), jnp.float32),
            pltpu.VMEM_SHARED((_N * _N + _N,), jnp.float32),
        ],
    )
    def build(e_bwd, e_fwd, zeros_in, ones_in, a_out, deg_out, src_v, dst_v,
              idx_v, ones_v, a_sh):
        cid = lax.axis_index("c")
        sid = lax.axis_index("s")
        row0 = sid * rows_per_tile
        ebase = sid * e_per_tile

        pltpu.sync_copy(zeros_in, a_sh.at[pl.ds(row0, rows_per_tile)])
        pltpu.sync_copy(ones_in, ones_v)

        @pl.when(sid == 0)
        def _():
            pltpu.sync_copy(zeros_in.at[pl.ds(0, _N)],
                            a_sh.at[pl.ds(_N * _N, _N)])

        def scatter_edges(e_ref):
            pltpu.sync_copy(e_ref.at[0, pl.ds(ebase, e_per_tile)], src_v)
            pltpu.sync_copy(e_ref.at[1, pl.ds(ebase, e_per_tile)], dst_v)

            def body(i, _):
                sl = pl.ds(i * lanes, lanes)
                idx_v[sl] = dst_v[sl] * _N + src_v[sl]
                src_v[sl] = dst_v[sl] + (_N * _N)
                return ()

            lax.fori_loop(0, e_per_tile // lanes, body, ())
            plsc.subcore_barrier()
            pltpu.sync_copy(ones_v, a_sh.at[idx_v], add=True)
            pltpu.sync_copy(ones_v, a_sh.at[src_v], add=True)
            plsc.subcore_barrier()

        @pl.when(cid == 0)
        def _():
            scatter_edges(e_bwd)
            pltpu.sync_copy(a_sh.at[pl.ds(row0, rows_per_tile)],
                            a_out.at[0, pl.ds(row0, rows_per_tile)])

            @pl.when(sid == 0)
            def _():
                pltpu.sync_copy(a_sh.at[pl.ds(_N * _N, _N)], deg_out.at[0])

        @pl.when(cid == 1)
        def _():
            scatter_edges(e_fwd)
            pltpu.sync_copy(a_sh.at[pl.ds(row0, rows_per_tile)],
                            a_out.at[1, pl.ds(row0, rows_per_tile)])

            @pl.when(sid == 0)
            def _():
                pltpu.sync_copy(a_sh.at[pl.ds(_N * _N, _N)], deg_out.at[1])

    a_flat, deg = build(edge_bwd, edge_fwd, zeros_hbm, ones_hbm)
    return a_flat.reshape(2, _N, _N), deg


_BD = _B * _D


def _bf(x):
    return x.astype(jnp.bfloat16)


def _tc_body(x_ref, a_ref, deg_ref, wg0_ref, bg0_ref, wg1_ref, bg1_ref,
             wc_ref, wl_ref, ob_ref, out_ref):
    h = pl.program_id(1)
    a = a_ref[0]
    inv = 1.0 / jnp.maximum(deg_ref[0], 1.0)
    xh = jnp.concatenate([x_ref[b, 0] for b in range(_B)], axis=1)
    m0 = jnp.dot(a, _bf(xh), preferred_element_type=jnp.float32)
    g0 = m0 * inv + xh

    h1_parts = []
    for b in range(_B):
        gb = g0[:, b * _D:(b + 1) * _D]
        h1_parts.append(jnp.maximum(
            jnp.dot(_bf(gb), wg0_ref[0], preferred_element_type=jnp.float32)
            + bg0_ref[0], 0.0))
    h1cat = jnp.concatenate(
        [p[:, :_D] for p in h1_parts] + [p[:, _D:] for p in h1_parts], axis=1)

    m1 = jnp.dot(a, _bf(h1cat), preferred_element_type=jnp.float32)
    g1 = m1 * inv + h1cat

    zs = []
    for b in range(_B):
        zb = None
        for c in range(_C):
            gcb = g1[:, (c * _B + b) * _D:(c * _B + b + 1) * _D]
            h2 = jnp.maximum(
                jnp.dot(_bf(gcb), wg1_ref[c, 0],
                        preferred_element_type=jnp.float32) + bg1_ref[c, 0],
                0.0)
            zc = jnp.dot(_bf(h2), wl_ref[c], preferred_element_type=jnp.float32)
            zb = zc if zb is None else zb + zc
        zs.append(zb)
    z = jnp.concatenate(zs, axis=1)

    y = jnp.dot(wc_ref[...], _bf(z), preferred_element_type=jnp.float32)

    @pl.when(h == 0)
    def _():
        for b in range(_B):
            out_ref[b, 0] = y[:, b * _D:(b + 1) * _D] + ob_ref[...]

    @pl.when(h == 1)
    def _():
        for b in range(_B):
            out_ref[b, 0] = out_ref[b, 0] + y[:, b * _D:(b + 1) * _D]


def kernel(inputs, edge_index_bwd, edge_index_fwd, W_gcn, b_gcn, W_conv,
           b_conv, W_lin, b_lin):
    a2, deg2 = _build_adjacency(edge_index_bwd, edge_index_fwd)
    a2 = a2.astype(jnp.bfloat16)
    deg2 = deg2.reshape(2, _N, 1)

    wg = W_gcn.astype(jnp.bfloat16)
    wg0 = jnp.concatenate([wg[0, :, 0], wg[1, :, 0]], axis=-1)
    bg0 = jnp.concatenate([b_gcn[0, :, 0], b_gcn[1, :, 0]], axis=-1)[:, None, :]
    wg1 = wg[:, :, 1]
    bg1 = b_gcn[:, :, 1][:, :, None, :]
    wc = W_conv.astype(jnp.bfloat16)
    wl = W_lin.reshape(_C, _D, _D).astype(jnp.bfloat16)
    ob = b_conv[:, None] * jnp.sum(W_lin, axis=0)[None, :] + b_lin[None, :]

    out = pl.pallas_call(
        _tc_body,
        grid=(_P, 2),
        in_specs=[
            pl.BlockSpec((_B, 1, _N, _D), lambda p, h: (0, 2 * p + h, 0, 0)),
            pl.BlockSpec((1, _N, _N), lambda p, h: (h, 0, 0)),
            pl.BlockSpec((1, _N, 1), lambda p, h: (h, 0, 0)),
            pl.BlockSpec((1, _D, 2 * _D), lambda p, h: (h, 0, 0)),
            pl.BlockSpec((1, 1, 2 * _D), lambda p, h: (h, 0, 0)),
            pl.BlockSpec((_C, 1, _D, _D), lambda p, h: (0, h, 0, 0)),
            pl.BlockSpec((_C, 1, 1, _D), lambda p, h: (0, h, 0, 0)),
            pl.BlockSpec((_N, _N), lambda p, h: (0, h)),
            pl.BlockSpec((_C, _D, _D), lambda p, h: (0, 0, 0)),
            pl.BlockSpec((_N, _D), lambda p, h: (0, 0)),
        ],
        out_specs=pl.BlockSpec((_B, 1, _N, _D), lambda p, h: (0, p, 0, 0)),
        out_shape=jax.ShapeDtypeStruct((_B, _P, _N, _D), jnp.float32),
        compiler_params=pltpu.CompilerParams(
            dimension_semantics=("parallel", "arbitrary"),
            vmem_limit_bytes=120 * 1024 * 1024,
        ),
    )(inputs, a2, deg2, wg0, bg0, wg1, bg1, wc, wl, ob)
    return out


# speedup vs baseline: 1.8763x; 1.0092x over previous
import functools

import jax
import jax.numpy as jnp
from jax import lax
from jax.experimental import pallas as pl
from jax.experimental.pallas import tpu as pltpu
from jax.experimental.pallas import tpu_sc as plsc

_B, _T, _N, _D = 4, 8, 1024, 128
_C, _L = 2, 2
_E = 32768
_P = 4


def _build_adjacency(edge_bwd, edge_fwd):
    info = plsc.get_sparse_core_info()
    n_sub = info.num_subcores
    lanes = info.num_lanes
    e_per_tile = _E // n_sub
    rows_per_tile = (_N * _N) // n_sub

    zeros_hbm = jnp.zeros((rows_per_tile,), jnp.float32)
    ones_hbm = jnp.ones((e_per_tile,), jnp.float32)

    mesh = plsc.VectorSubcoreMesh(core_axis_name="c", subcore_axis_name="s")

    @functools.partial(
        pl.kernel,
        mesh=mesh,
        out_type=[
            jax.ShapeDtypeStruct((2, _N * _N), jnp.float32),
            jax.ShapeDtypeStruct((2, _N), jnp.float32),
        ],
        scratch_types=[
            pltpu.VMEM((e_per_tile,), jnp.int32),
            pltpu.VMEM((e_per_tile,), jnp.int32),
            pltpu.VMEM((e_per_tile,), jnp.int32),
            pltpu.VMEM((e_per_tile,), jnp.float32),
            pltpu.VMEM_SHARED((_N * _N + _N,), jnp.float32),
        ],
    )
    def build(e_bwd, e_fwd, zeros_in, ones_in, a_out, deg_out, src_v, dst_v,
              idx_v, ones_v, a_sh):
        cid = lax.axis_index("c")
        sid = lax.axis_index("s")
        row0 = sid * rows_per_tile
        ebase = sid * e_per_tile

        pltpu.sync_copy(zeros_in, a_sh.at[pl.ds(row0, rows_per_tile)])
        pltpu.sync_copy(ones_in, ones_v)

        @pl.when(sid == 0)
        def _():
            pltpu.sync_copy(zeros_in.at[pl.ds(0, _N)],
                            a_sh.at[pl.ds(_N * _N, _N)])

        def scatter_edges(e_ref):
            pltpu.sync_copy(e_ref.at[0, pl.ds(ebase, e_per_tile)], src_v)
            pltpu.sync_copy(e_ref.at[1, pl.ds(ebase, e_per_tile)], dst_v)

            def body(i, _):
                sl = pl.ds(i * lanes, lanes)
                idx_v[sl] = dst_v[sl] * _N + src_v[sl]
                src_v[sl] = dst_v[sl] + (_N * _N)
                return ()

            lax.fori_loop(0, e_per_tile // lanes, body, ())
            plsc.subcore_barrier()
            pltpu.sync_copy(ones_v, a_sh.at[idx_v], add=True)
            pltpu.sync_copy(ones_v, a_sh.at[src_v], add=True)
            plsc.subcore_barrier()

        @pl.when(cid == 0)
        def _():
            scatter_edges(e_bwd)
            pltpu.sync_copy(a_sh.at[pl.ds(row0, rows_per_tile)],
                            a_out.at[0, pl.ds(row0, rows_per_tile)])

            @pl.when(sid == 0)
            def _():
                pltpu.sync_copy(a_sh.at[pl.ds(_N * _N, _N)], deg_out.at[0])

        @pl.when(cid == 1)
        def _():
            scatter_edges(e_fwd)
            pltpu.sync_copy(a_sh.at[pl.ds(row0, rows_per_tile)],
                            a_out.at[1, pl.ds(row0, rows_per_tile)])

            @pl.when(sid == 0)
            def _():
                pltpu.sync_copy(a_sh.at[pl.ds(_N * _N, _N)], deg_out.at[1])

    a_flat, deg = build(edge_bwd, edge_fwd, zeros_hbm, ones_hbm)
    return a_flat.reshape(2, _N, _N), deg


_BD = _B * _D


def _bf(x):
    return x.astype(jnp.bfloat16)


def _tc_body(x_ref, a_ref, deg_ref, wg0_ref, bg0_ref, wg1_ref, bg1_ref,
             wc_ref, wl_ref, ob_ref, out_ref, abar_ref):
    p = pl.program_id(0)

    @pl.when(p == 0)
    def _():
        for hh in range(2):
            inv = 1.0 / jnp.maximum(deg_ref[hh], 1.0)
            abar_ref[hh] = _bf(a_ref[hh].astype(jnp.float32) * inv)

    ys = []
    for hh in range(2):
        ab = abar_ref[hh]
        xh = jnp.concatenate([x_ref[b, hh] for b in range(_B)], axis=1)
        m0 = jnp.dot(ab, _bf(xh), preferred_element_type=jnp.float32)
        g0 = m0 + xh

        h1_parts = []
        for b in range(_B):
            gb = g0[:, b * _D:(b + 1) * _D]
            h1_parts.append(jnp.maximum(
                jnp.dot(_bf(gb), wg0_ref[hh],
                        preferred_element_type=jnp.float32) + bg0_ref[hh],
                0.0))
        h1cat = jnp.concatenate(
            [q[:, :_D] for q in h1_parts] + [q[:, _D:] for q in h1_parts],
            axis=1)

        m1 = jnp.dot(ab, _bf(h1cat), preferred_element_type=jnp.float32)
        g1 = m1 + h1cat

        zs = []
        for b in range(_B):
            zb = None
            for c in range(_C):
                gcb = g1[:, (c * _B + b) * _D:(c * _B + b + 1) * _D]
                h2 = jnp.maximum(
                    jnp.dot(_bf(gcb), wg1_ref[c, hh],
                            preferred_element_type=jnp.float32)
                    + bg1_ref[c, hh], 0.0)
                zc = jnp.dot(_bf(h2), wl_ref[c],
                             preferred_element_type=jnp.float32)
                zb = zc if zb is None else zb + zc
            zs.append(zb)
        z = jnp.concatenate(zs, axis=1)

        ys.append(jnp.dot(wc_ref[:, hh * _N:(hh + 1) * _N], _bf(z),
                          preferred_element_type=jnp.float32))
    yt = ys[0] + ys[1]
    for b in range(_B):
        out_ref[b, 0] = yt[:, b * _D:(b + 1) * _D] + ob_ref[...]


def kernel(inputs, edge_index_bwd, edge_index_fwd, W_gcn, b_gcn, W_conv,
           b_conv, W_lin, b_lin):
    a2, deg2 = _build_adjacency(edge_index_bwd, edge_index_fwd)
    a2 = a2.astype(jnp.bfloat16)
    deg2 = deg2.reshape(2, _N, 1)

    wg = W_gcn.astype(jnp.bfloat16)
    wg0 = jnp.concatenate([wg[0, :, 0], wg[1, :, 0]], axis=-1)
    bg0 = jnp.concatenate([b_gcn[0, :, 0], b_gcn[1, :, 0]], axis=-1)[:, None, :]
    wg1 = wg[:, :, 1]
    bg1 = b_gcn[:, :, 1][:, :, None, :]
    wc = W_conv.astype(jnp.bfloat16)
    wl = W_lin.reshape(_C, _D, _D).astype(jnp.bfloat16)
    ob = b_conv[:, None] * jnp.sum(W_lin, axis=0)[None, :] + b_lin[None, :]

    out = pl.pallas_call(
        _tc_body,
        grid=(_P,),
        in_specs=[
            pl.BlockSpec((_B, 2, _N, _D), lambda p: (0, p, 0, 0)),
            pl.BlockSpec((2, _N, _N), lambda p: (0, 0, 0)),
            pl.BlockSpec((2, _N, 1), lambda p: (0, 0, 0)),
            pl.BlockSpec((2, _D, 2 * _D), lambda p: (0, 0, 0)),
            pl.BlockSpec((2, 1, 2 * _D), lambda p: (0, 0, 0)),
            pl.BlockSpec((_C, 2, _D, _D), lambda p: (0, 0, 0, 0)),
            pl.BlockSpec((_C, 2, 1, _D), lambda p: (0, 0, 0, 0)),
            pl.BlockSpec((_N, 2 * _N), lambda p: (0, 0)),
            pl.BlockSpec((_C, _D, _D), lambda p: (0, 0, 0)),
            pl.BlockSpec((_N, _D), lambda p: (0, 0)),
        ],
        out_specs=pl.BlockSpec((_B, 1, _N, _D), lambda p: (0, p, 0, 0)),
        out_shape=jax.ShapeDtypeStruct((_B, _P, _N, _D), jnp.float32),
        scratch_shapes=[pltpu.VMEM((2, _N, _N), jnp.bfloat16)],
        compiler_params=pltpu.CompilerParams(
            dimension_semantics=("arbitrary",),
            vmem_limit_bytes=120 * 1024 * 1024,
        ),
    )(inputs, a2, deg2, wg0, bg0, wg1, bg1, wc, wl, ob)
    return out


# speedup vs baseline: 2.0060x; 1.0691x over previous
import functools

import jax
import jax.numpy as jnp
from jax import lax
from jax.experimental import pallas as pl
from jax.experimental.pallas import tpu as pltpu
from jax.experimental.pallas import tpu_sc as plsc

_B, _T, _N, _D = 4, 8, 1024, 128
_C, _L = 2, 2
_E = 32768
_P = 4


def _build_adjacency(edge_bwd, edge_fwd):
    info = plsc.get_sparse_core_info()
    n_sub = info.num_subcores
    lanes = info.num_lanes
    e_per_tile = _E // n_sub
    rows_per_tile = (_N * _N) // n_sub

    n_per_tile = _N // n_sub

    zeros_hbm = jnp.zeros((rows_per_tile,), jnp.float32)
    ones_hbm = jnp.ones((2 * e_per_tile,), jnp.float32)

    mesh = plsc.VectorSubcoreMesh(core_axis_name="c", subcore_axis_name="s")

    @functools.partial(
        pl.kernel,
        mesh=mesh,
        out_type=[
            jax.ShapeDtypeStruct((2, _N, _N), jnp.float32),
            jax.ShapeDtypeStruct((2, _N), jnp.float32),
        ],
        scratch_types=[
            pltpu.VMEM((e_per_tile,), jnp.int32),
            pltpu.VMEM((e_per_tile,), jnp.int32),
            pltpu.VMEM((2 * e_per_tile,), jnp.int32),
            pltpu.VMEM((2 * e_per_tile,), jnp.float32),
            pltpu.VMEM_SHARED((_N * _N + _N,), jnp.float32),
            pltpu.SemaphoreType.DMA,
            pltpu.SemaphoreType.DMA,
        ],
    )
    def build(e_bwd, e_fwd, zeros_in, ones_in, a_out, deg_out, src_v, dst_v,
              idx_v, ones_v, a_sh, zsem, wsem):
        cid = lax.axis_index("c")
        sid = lax.axis_index("s")
        row0 = sid * rows_per_tile
        ebase = sid * e_per_tile

        zcopy = pltpu.async_copy(zeros_in, a_sh.at[pl.ds(row0, rows_per_tile)],
                                 zsem)
        pltpu.sync_copy(ones_in, ones_v)

        @pl.when(sid == 0)
        def _():
            pltpu.sync_copy(zeros_in.at[pl.ds(0, _N)],
                            a_sh.at[pl.ds(_N * _N, _N)])

        def scatter_edges(e_ref):
            pltpu.sync_copy(e_ref.at[0, pl.ds(ebase, e_per_tile)], src_v)
            pltpu.sync_copy(e_ref.at[1, pl.ds(ebase, e_per_tile)], dst_v)

            def body(i, _):
                sl = pl.ds(i * lanes, lanes)
                idx_v[sl] = dst_v[sl] * _N + src_v[sl]
                idx_v[pl.ds(e_per_tile + i * lanes, lanes)] = (
                    dst_v[sl] + (_N * _N))
                return ()

            lax.fori_loop(0, e_per_tile // lanes, body, ())
            zcopy.wait()
            plsc.subcore_barrier()
            pltpu.sync_copy(ones_v, a_sh.at[idx_v], add=True)
            plsc.subcore_barrier()

        def write_out(half):
            handles = [
                pltpu.async_copy(a_sh.at[pl.ds(row0 + r * _N, _N)],
                                 a_out.at[half, sid * n_per_tile + r, :], wsem)
                for r in range(n_per_tile)
            ]
            for hdl in handles:
                hdl.wait()

            @pl.when(sid == 0)
            def _():
                pltpu.sync_copy(a_sh.at[pl.ds(_N * _N, _N)], deg_out.at[half])

        @pl.when(cid == 0)
        def _():
            scatter_edges(e_bwd)
            write_out(0)

        @pl.when(cid == 1)
        def _():
            scatter_edges(e_fwd)
            write_out(1)

    return build(edge_bwd, edge_fwd, zeros_hbm, ones_hbm)


_BD = _B * _D


def _bf(x):
    return x.astype(jnp.bfloat16)


def _tc_body(x_ref, a_ref, deg_ref, wg0_ref, bg0_ref, wg1_ref, bg1_ref,
             wc_ref, wl_ref, ob_ref, out_ref, abar_ref):
    p = pl.program_id(0)

    @pl.when(p == 0)
    def _():
        for hh in range(2):
            inv = 1.0 / jnp.maximum(deg_ref[hh], 1.0)
            abar_ref[hh] = _bf(a_ref[hh] * inv)

    ys = []
    for hh in range(2):
        ab = abar_ref[hh]
        xh = jnp.concatenate([x_ref[b, hh] for b in range(_B)], axis=1)
        m0 = jnp.dot(ab, _bf(xh), preferred_element_type=jnp.float32)
        g0 = m0 + xh

        h1_parts = []
        for b in range(_B):
            gb = g0[:, b * _D:(b + 1) * _D]
            h1_parts.append(jnp.maximum(
                jnp.dot(_bf(gb), wg0_ref[hh],
                        preferred_element_type=jnp.float32) + bg0_ref[hh],
                0.0))
        h1cat = jnp.concatenate(
            [q[:, :_D] for q in h1_parts] + [q[:, _D:] for q in h1_parts],
            axis=1)

        m1 = jnp.dot(ab, _bf(h1cat), preferred_element_type=jnp.float32)
        g1 = m1 + h1cat

        zs = []
        for b in range(_B):
            zb = None
            for c in range(_C):
                gcb = g1[:, (c * _B + b) * _D:(c * _B + b + 1) * _D]
                h2 = jnp.maximum(
                    jnp.dot(_bf(gcb), wg1_ref[c, hh],
                            preferred_element_type=jnp.float32)
                    + bg1_ref[c, hh], 0.0)
                zc = jnp.dot(_bf(h2), wl_ref[c],
                             preferred_element_type=jnp.float32)
                zb = zc if zb is None else zb + zc
            zs.append(zb)
        z = jnp.concatenate(zs, axis=1)

        ys.append(jnp.dot(wc_ref[:, hh * _N:(hh + 1) * _N], _bf(z),
                          preferred_element_type=jnp.float32))
    yt = ys[0] + ys[1]
    for b in range(_B):
        out_ref[b, 0] = yt[:, b * _D:(b + 1) * _D] + ob_ref[...]


def kernel(inputs, edge_index_bwd, edge_index_fwd, W_gcn, b_gcn, W_conv,
           b_conv, W_lin, b_lin):
    a2, deg2 = _build_adjacency(edge_index_bwd, edge_index_fwd)
    deg2 = deg2.reshape(2, _N, 1)

    wg = W_gcn.astype(jnp.bfloat16)
    wg0 = jnp.concatenate([wg[0, :, 0], wg[1, :, 0]], axis=-1)
    bg0 = jnp.concatenate([b_gcn[0, :, 0], b_gcn[1, :, 0]], axis=-1)[:, None, :]
    wg1 = wg[:, :, 1]
    bg1 = b_gcn[:, :, 1][:, :, None, :]
    wc = W_conv.astype(jnp.bfloat16)
    wl = W_lin.reshape(_C, _D, _D).astype(jnp.bfloat16)
    ob = b_conv[:, None] * jnp.sum(W_lin, axis=0)[None, :] + b_lin[None, :]

    out = pl.pallas_call(
        _tc_body,
        grid=(_P,),
        in_specs=[
            pl.BlockSpec((_B, 2, _N, _D), lambda p: (0, p, 0, 0)),
            pl.BlockSpec((2, _N, _N), lambda p: (0, 0, 0)),
            pl.BlockSpec((2, _N, 1), lambda p: (0, 0, 0)),
            pl.BlockSpec((2, _D, 2 * _D), lambda p: (0, 0, 0)),
            pl.BlockSpec((2, 1, 2 * _D), lambda p: (0, 0, 0)),
            pl.BlockSpec((_C, 2, _D, _D), lambda p: (0, 0, 0, 0)),
            pl.BlockSpec((_C, 2, 1, _D), lambda p: (0, 0, 0, 0)),
            pl.BlockSpec((_N, 2 * _N), lambda p: (0, 0)),
            pl.BlockSpec((_C, _D, _D), lambda p: (0, 0, 0)),
            pl.BlockSpec((_N, _D), lambda p: (0, 0)),
        ],
        out_specs=pl.BlockSpec((_B, 1, _N, _D), lambda p: (0, p, 0, 0)),
        out_shape=jax.ShapeDtypeStruct((_B, _P, _N, _D), jnp.float32),
        scratch_shapes=[pltpu.VMEM((2, _N, _N), jnp.bfloat16)],
        compiler_params=pltpu.CompilerParams(
            dimension_semantics=("arbitrary",),
            vmem_limit_bytes=120 * 1024 * 1024,
        ),
    )(inputs, a2, deg2, wg0, bg0, wg1, bg1, wc, wl, ob)
    return out


# speedup vs baseline: 2.0062x; 1.0001x over previous
import functools

import jax
import jax.numpy as jnp
from jax import lax
from jax.experimental import pallas as pl
from jax.experimental.pallas import tpu as pltpu
from jax.experimental.pallas import tpu_sc as plsc

_B, _T, _N, _D = 4, 8, 1024, 128
_C, _L = 2, 2
_E = 32768
_P = 4


def _build_adjacency(edge_bwd, edge_fwd):
    info = plsc.get_sparse_core_info()
    n_sub = info.num_subcores
    lanes = info.num_lanes
    e_per_tile = _E // n_sub
    rows_per_tile = (_N * _N) // n_sub

    n_per_tile = _N // n_sub

    zeros_hbm = jnp.zeros((rows_per_tile,), jnp.float32)
    ones_hbm = jnp.ones((2 * e_per_tile,), jnp.float32)

    mesh = plsc.VectorSubcoreMesh(core_axis_name="c", subcore_axis_name="s")

    @functools.partial(
        pl.kernel,
        mesh=mesh,
        out_type=[
            jax.ShapeDtypeStruct((2, _N, _N), jnp.float32),
            jax.ShapeDtypeStruct((2, _N), jnp.float32),
        ],
        scratch_types=[
            pltpu.VMEM((e_per_tile,), jnp.int32),
            pltpu.VMEM((e_per_tile,), jnp.int32),
            pltpu.VMEM((2 * e_per_tile,), jnp.int32),
            pltpu.VMEM((2 * e_per_tile,), jnp.float32),
            pltpu.VMEM_SHARED((_N * _N + _N,), jnp.float32),
            pltpu.SemaphoreType.DMA,
            pltpu.SemaphoreType.DMA,
        ],
    )
    def build(e_bwd, e_fwd, zeros_in, ones_in, a_out, deg_out, src_v, dst_v,
              idx_v, ones_v, a_sh, zsem, wsem):
        cid = lax.axis_index("c")
        sid = lax.axis_index("s")
        row0 = sid * rows_per_tile
        ebase = sid * e_per_tile

        zcopy = pltpu.async_copy(zeros_in, a_sh.at[pl.ds(row0, rows_per_tile)],
                                 zsem)
        pltpu.sync_copy(ones_in, ones_v)

        @pl.when(sid == 0)
        def _():
            pltpu.sync_copy(zeros_in.at[pl.ds(0, _N)],
                            a_sh.at[pl.ds(_N * _N, _N)])

        def scatter_edges(e_ref):
            pltpu.sync_copy(e_ref.at[0, pl.ds(ebase, e_per_tile)], src_v)
            pltpu.sync_copy(e_ref.at[1, pl.ds(ebase, e_per_tile)], dst_v)

            def body(i, _):
                sl = pl.ds(i * lanes, lanes)
                idx_v[sl] = dst_v[sl] * _N + src_v[sl]
                idx_v[pl.ds(e_per_tile + i * lanes, lanes)] = (
                    dst_v[sl] + (_N * _N))
                return ()

            lax.fori_loop(0, e_per_tile // lanes, body, ())
            zcopy.wait()
            plsc.subcore_barrier()
            pltpu.sync_copy(ones_v, a_sh.at[idx_v], add=True)
            plsc.subcore_barrier()

        def write_out(half):
            handles = [
                pltpu.async_copy(a_sh.at[pl.ds(row0 + r * _N, _N)],
                                 a_out.at[half, sid * n_per_tile + r, :], wsem)
                for r in range(n_per_tile)
            ]
            for hdl in handles:
                hdl.wait()

            @pl.when(sid == 0)
            def _():
                pltpu.sync_copy(a_sh.at[pl.ds(_N * _N, _N)], deg_out.at[half])

        @pl.when(cid == 0)
        def _():
            scatter_edges(e_bwd)
            write_out(0)

        @pl.when(cid == 1)
        def _():
            scatter_edges(e_fwd)
            write_out(1)

    return build(edge_bwd, edge_fwd, zeros_hbm, ones_hbm)


_BD = _B * _D


def _bf(x):
    return x.astype(jnp.bfloat16)


def _tc_body(x_ref, a_ref, deg_ref, wg0_ref, bg0_ref, wg1_ref, bg1_ref,
             wc_ref, wl_ref, ob_ref, out_ref, abar_ref, xs_ref, g0_ref,
             h1_ref, g1_ref, z_ref):
    p = pl.program_id(0)

    @pl.when(p == 0)
    def _():
        for hh in range(2):
            inv = 1.0 / jnp.maximum(deg_ref[hh], 1.0)
            abar_ref[hh] = _bf(a_ref[hh] * inv)

    yt = None
    for hh in range(2):
        ab = abar_ref[hh]
        for b in range(_B):
            xs_ref[:, b * _D:(b + 1) * _D] = _bf(x_ref[b, hh])
        m0 = jnp.dot(ab, xs_ref[...], preferred_element_type=jnp.float32)
        for b in range(_B):
            sl = pl.ds(b * _D, _D)
            g0_ref[:, sl] = _bf(m0[:, b * _D:(b + 1) * _D] + x_ref[b, hh])

        for b in range(_B):
            gb = g0_ref[:, b * _D:(b + 1) * _D]
            h1_ref[:, b * 2 * _D:(b + 1) * 2 * _D] = _bf(jnp.maximum(
                jnp.dot(gb, wg0_ref[hh],
                        preferred_element_type=jnp.float32)
                + bg0_ref[hh], 0.0))

        m1 = jnp.dot(ab, h1_ref[...], preferred_element_type=jnp.float32)
        g1_ref[...] = _bf(m1) + h1_ref[...]

        for b in range(_B):
            zb = None
            for c in range(_C):
                gcb = g1_ref[:, (2 * b + c) * _D:(2 * b + c + 1) * _D]
                h2 = _bf(jnp.maximum(
                    jnp.dot(gcb, wg1_ref[c, hh],
                            preferred_element_type=jnp.float32)
                    + bg1_ref[c, hh], 0.0))
                zc = jnp.dot(h2, wl_ref[c],
                             preferred_element_type=jnp.float32)
                zb = zc if zb is None else zb + zc
            z_ref[:, b * _D:(b + 1) * _D] = _bf(zb)

        y = jnp.dot(wc_ref[:, hh * _N:(hh + 1) * _N], z_ref[...],
                    preferred_element_type=jnp.float32)
        yt = y if yt is None else yt + y
    for b in range(_B):
        out_ref[b, 0] = yt[:, b * _D:(b + 1) * _D] + ob_ref[...]


def kernel(inputs, edge_index_bwd, edge_index_fwd, W_gcn, b_gcn, W_conv,
           b_conv, W_lin, b_lin):
    a2, deg2 = _build_adjacency(edge_index_bwd, edge_index_fwd)
    deg2 = deg2.reshape(2, _N, 1)

    wg = W_gcn.astype(jnp.bfloat16)
    wg0 = jnp.concatenate([wg[0, :, 0], wg[1, :, 0]], axis=-1)
    bg0 = jnp.concatenate([b_gcn[0, :, 0], b_gcn[1, :, 0]], axis=-1)[:, None, :]
    wg1 = wg[:, :, 1]
    bg1 = b_gcn[:, :, 1][:, :, None, :]
    wc = W_conv.astype(jnp.bfloat16)
    wl = W_lin.reshape(_C, _D, _D).astype(jnp.bfloat16)
    ob = b_conv[:, None] * jnp.sum(W_lin, axis=0)[None, :] + b_lin[None, :]

    out = pl.pallas_call(
        _tc_body,
        grid=(_P,),
        in_specs=[
            pl.BlockSpec((_B, 2, _N, _D), lambda p: (0, p, 0, 0)),
            pl.BlockSpec((2, _N, _N), lambda p: (0, 0, 0)),
            pl.BlockSpec((2, _N, 1), lambda p: (0, 0, 0)),
            pl.BlockSpec((2, _D, 2 * _D), lambda p: (0, 0, 0)),
            pl.BlockSpec((2, 1, 2 * _D), lambda p: (0, 0, 0)),
            pl.BlockSpec((_C, 2, _D, _D), lambda p: (0, 0, 0, 0)),
            pl.BlockSpec((_C, 2, 1, _D), lambda p: (0, 0, 0, 0)),
            pl.BlockSpec((_N, 2 * _N), lambda p: (0, 0)),
            pl.BlockSpec((_C, _D, _D), lambda p: (0, 0, 0)),
            pl.BlockSpec((_N, _D), lambda p: (0, 0)),
        ],
        out_specs=pl.BlockSpec((_B, 1, _N, _D), lambda p: (0, p, 0, 0)),
        out_shape=jax.ShapeDtypeStruct((_B, _P, _N, _D), jnp.float32),
        scratch_shapes=[
            pltpu.VMEM((2, _N, _N), jnp.bfloat16),
            pltpu.VMEM((_N, _BD), jnp.bfloat16),
            pltpu.VMEM((_N, _BD), jnp.bfloat16),
            pltpu.VMEM((_N, 2 * _BD), jnp.bfloat16),
            pltpu.VMEM((_N, 2 * _BD), jnp.bfloat16),
            pltpu.VMEM((_N, _BD), jnp.bfloat16),
        ],
        compiler_params=pltpu.CompilerParams(
            dimension_semantics=("arbitrary",),
            vmem_limit_bytes=120 * 1024 * 1024,
        ),
    )(inputs, a2, deg2, wg0, bg0, wg1, bg1, wc, wl, ob)
    return out
